# baseline (speedup 1.0000x reference)
import functools

import jax
import jax.numpy as jnp
from jax import lax
from jax.experimental import pallas as pl
from jax.experimental.pallas import tpu as pltpu
from jax.experimental.pallas import tpu_sc as plsc

N = 1_000_000
NPAD = 1_048_576
NW = 32
C = NPAD // NW
S = 2048
G = 128
B = 2048
TRASH = NPAD
X_LO, X_HI = -1.0, 1.0
T_LO, T_HI = 0.0, 1.0

_mesh = plsc.VectorSubcoreMesh(core_axis_name="c", subcore_axis_name="s")


def _wid():
    return lax.axis_index("s") * 2 + lax.axis_index("c")


@functools.partial(
    pl.kernel,
    out_type=(
        jax.ShapeDtypeStruct((NPAD,), jnp.int32),
        jax.ShapeDtypeStruct((NW, 16), jnp.int32),
    ),
    mesh=_mesh,
    compiler_params=pltpu.CompilerParams(needs_layout_passes=False),
    scratch_types=[
        pltpu.VMEM((S,), jnp.float32),
        pltpu.VMEM((16,), jnp.float32),
        pltpu.VMEM((C + 16,), jnp.int32),
        pltpu.VMEM((16,), jnp.int32),
    ],
)
def _k2_compact(loss_hbm, mean_hbm, kidx_hbm, counts_hbm,
                loss_v, mean_v, kidx_v, cnt_v):
    w = _wid()
    base = w * C
    pltpu.sync_copy(mean_hbm, mean_v)
    meanv = mean_v[...]
    lane = lax.iota(jnp.int32, 16)

    def outer(f, wp):
        pltpu.sync_copy(loss_hbm.at[pl.ds(base + f * S, S)], loss_v)

        def inner(p, wpv):
            lv = loss_v[pl.ds(p * 16, 16)]
            m = lv > meanv
            gi = (base + f * S + p * 16) + lane
            pos = plsc.cumsum(m.astype(jnp.int32))
            plsc.store_scatter(kidx_v, [wpv + pos - 1], gi, mask=m)
            return wpv + plsc.all_reduce_population_count(m)

        return lax.fori_loop(0, S // 16, inner, wp)

    wp = lax.fori_loop(0, C // S, outer, jnp.zeros((16,), jnp.int32))
    pltpu.sync_copy(kidx_v.at[pl.ds(0, C)], kidx_hbm.at[pl.ds(base, C)])
    cnt_v[...] = wp
    pltpu.sync_copy(cnt_v, counts_hbm.at[w])


@functools.partial(
    pl.kernel,
    out_type=(
        jax.ShapeDtypeStruct((NPAD + 16,), jnp.float32),
        jax.ShapeDtypeStruct((NPAD + 16,), jnp.float32),
    ),
    mesh=_mesh,
    compiler_params=pltpu.CompilerParams(needs_layout_passes=False),
    scratch_types=[
        pltpu.VMEM((NW, 16), jnp.int32),
        pltpu.VMEM((B,), jnp.int32),
        pltpu.VMEM((B // G, G), jnp.int32),
        pltpu.VMEM((B // G, G), jnp.int32),
        pltpu.VMEM((B // G, G), jnp.float32),
        pltpu.VMEM((B // G, G), jnp.float32),
        pltpu.SemaphoreType.DMA,
    ],
)
def _k3_assemble(x_hbm, t_hbm, xn_hbm, tn_hbm, kidx_hbm, counts_hbm,
                 xo_hbm, to_hbm,
                 counts_v, kbuf, sidx, oidx, xbuf, tbuf, sem):
    w = _wid()
    base = w * C
    lane = lax.iota(jnp.int32, 16)
    zeros = jnp.zeros((16,), jnp.int32)

    pltpu.sync_copy(counts_hbm, counts_v)
    c0 = plsc.load_gather(counts_v, [lane, zeros])
    c1 = plsc.load_gather(counts_v, [lane + 16, zeros])
    s0 = plsc.cumsum(c0)
    s1 = plsc.cumsum(c1)
    tot0 = jnp.max(s0)
    total = tot0 + jnp.max(s1)
    e0 = s0 - c0
    e1 = (s1 - c1) + tot0
    lsel = jnp.where(w < 16, w, w - 16)
    ew = jnp.where(w < 16, e0, e1)
    cwv = jnp.where(w < 16, c0, c1)
    p_w = jnp.sum(jnp.where(lane == lsel, ew, 0))
    c_w = jnp.sum(jnp.where(lane == lsel, cwv, 0))

    def keep_body(q, _):
        @pl.when(q * B < c_w)
        def _():
            pltpu.sync_copy(kidx_hbm.at[pl.ds(base + q * B, B)], kbuf)
            for r in range(B // G):
                for p in range(G // 16):
                    i = r * G + p * 16
                    kv = kbuf[pl.ds(i, 16)]
                    pos = (q * B + i) + lane
                    valid = pos < c_w
                    sidx[r, pl.ds(p * 16, 16)] = jnp.where(valid, kv, 0)
                    oidx[r, pl.ds(p * 16, 16)] = jnp.where(
                        valid, p_w + pos, TRASH + lane)
            gds = []
            for r in range(B // G):
                gds.append(pltpu.async_copy(x_hbm.at[sidx.at[r]],
                                            xbuf.at[r], sem))
                gds.append(pltpu.async_copy(t_hbm.at[sidx.at[r]],
                                            tbuf.at[r], sem))
            for d in gds:
                d.wait()
            wds = []
            for r in range(B // G):
                wds.append(pltpu.async_copy(xbuf.at[r],
                                            xo_hbm.at[oidx.at[r]], sem))
                wds.append(pltpu.async_copy(tbuf.at[r],
                                            to_hbm.at[oidx.at[r]], sem))
            for d in wds:
                d.wait()
        return 0

    lax.fori_loop(0, C // B, keep_body, 0)

    def tail_body(q, _):
        start = base + q * B

        @pl.when(start + B > total)
        def _():
            for r in range(B // G):
                for p in range(G // 16):
                    j = (start + r * G + p * 16) + lane
                    tv = j >= total
                    sidx[r, pl.ds(p * 16, 16)] = jnp.clip(j - total, 0, N - 1)
                    oidx[r, pl.ds(p * 16, 16)] = jnp.where(tv, j, TRASH + lane)
            gds = []
            for r in range(B // G):
                gds.append(pltpu.async_copy(xn_hbm.at[sidx.at[r]],
                                            xbuf.at[r], sem))
                gds.append(pltpu.async_copy(tn_hbm.at[sidx.at[r]],
                                            tbuf.at[r], sem))
            for d in gds:
                d.wait()
            wds = []
            for r in range(B // G):
                wds.append(pltpu.async_copy(xbuf.at[r],
                                            xo_hbm.at[oidx.at[r]], sem))
                wds.append(pltpu.async_copy(tbuf.at[r],
                                            to_hbm.at[oidx.at[r]], sem))
            for d in wds:
                d.wait()
        return 0

    lax.fori_loop(0, C // B, tail_body, 0)


def kernel(loss, x, t):
    mean = jnp.mean(loss)
    mean_arr = jnp.full((16,), mean, jnp.float32)
    lf = jnp.pad(loss.reshape(-1), (0, NPAD - N), constant_values=-1.0)
    kn = jax.random.split(jax.random.key(1), 2)
    xn = jax.random.uniform(kn[0], (N, 1), minval=X_LO, maxval=X_HI,
                            dtype=jnp.float32).reshape(-1)
    tn = jax.random.uniform(kn[1], (N, 1), minval=T_LO, maxval=T_HI,
                            dtype=jnp.float32).reshape(-1)
    kidx, counts = _k2_compact(lf, mean_arr)
    xo, to = _k3_assemble(x.reshape(-1), t.reshape(-1), xn, tn, kidx, counts)
    return (xo[:N, None], to[:N, None])


# speedup vs baseline: 4.1929x; 4.1929x over previous
import functools

import jax
import jax.numpy as jnp
from jax import lax
from jax.experimental import pallas as pl
from jax.experimental.pallas import tpu as pltpu
from jax.experimental.pallas import tpu_sc as plsc

N = 1_000_000
NPAD = 1_048_576
NW = 32
C = NPAD // NW
S = 8192
G = 128
B = 2048
NTR = 17
TRASH = NPAD
X_LO, X_HI = -1.0, 1.0
T_LO, T_HI = 0.0, 1.0

_mesh = plsc.VectorSubcoreMesh(core_axis_name="c", subcore_axis_name="s")


def _wid():
    return lax.axis_index("s") * 2 + lax.axis_index("c")


@functools.partial(
    pl.kernel,
    out_type=jax.ShapeDtypeStruct((NW, 16), jnp.int32),
    mesh=_mesh,
    compiler_params=pltpu.CompilerParams(needs_layout_passes=False),
    scratch_types=[
        pltpu.VMEM((S,), jnp.float32),
        pltpu.VMEM((16,), jnp.float32),
        pltpu.VMEM((16,), jnp.int32),
    ],
)
def _k1_count(loss_hbm, mean_hbm, counts_hbm, loss_v, mean_v, cnt_v):
    w = _wid()
    base = w * C
    pltpu.sync_copy(mean_hbm, mean_v)
    meanv = mean_v[...]

    def outer(f, wp):
        pltpu.sync_copy(loss_hbm.at[pl.ds(base + f * S, S)], loss_v)

        def inner(p, wpv):
            m = loss_v[pl.ds(p * 16, 16)] > meanv
            return wpv + plsc.all_reduce_population_count(m)

        return lax.fori_loop(0, S // 16, inner, wp)

    wp = lax.fori_loop(0, C // S, outer, jnp.zeros((16,), jnp.int32))
    cnt_v[...] = wp
    pltpu.sync_copy(cnt_v, counts_hbm.at[w])


@functools.partial(
    pl.kernel,
    out_type=(
        jax.ShapeDtypeStruct((NPAD + 16,), jnp.float32),
        jax.ShapeDtypeStruct((NPAD + 16,), jnp.float32),
    ),
    mesh=_mesh,
    compiler_params=pltpu.CompilerParams(needs_layout_passes=False),
    scratch_types=[
        pltpu.VMEM((NW, 16), jnp.int32),
        pltpu.VMEM((S,), jnp.float32),
        pltpu.VMEM((S,), jnp.float32),
        pltpu.VMEM((S,), jnp.float32),
        pltpu.VMEM((16,), jnp.float32),
        pltpu.VMEM((C + 2 * B,), jnp.float32),
        pltpu.VMEM((C + 2 * B,), jnp.float32),
        pltpu.VMEM((16,), jnp.float32),
        pltpu.VMEM((16,), jnp.float32),
        pltpu.VMEM((NTR, G), jnp.int32),
        pltpu.VMEM((NTR, G), jnp.float32),
        pltpu.VMEM((NTR, G), jnp.float32),
        pltpu.VMEM((B + 16,), jnp.float32),
        pltpu.VMEM((B + 16,), jnp.float32),
        pltpu.VMEM((B,), jnp.float32),
        pltpu.VMEM((B,), jnp.float32),
        pltpu.SemaphoreType.DMA,
    ],
)
def _k2_place(loss_hbm, x_hbm, t_hbm, xn_hbm, tn_hbm, mean_hbm, counts_hbm,
              xo_hbm, to_hbm,
              counts_v, loss_v, x_v, t_v, mean_v, xk_v, tk_v,
              headx, headt, ridx, rdx, rdt, winx, wint, xob, tob, sem):
    w = _wid()
    base = w * C
    lane = lax.iota(jnp.int32, 16)
    zeros = jnp.zeros((16,), jnp.int32)

    pltpu.sync_copy(mean_hbm, mean_v)
    pltpu.sync_copy(counts_hbm, counts_v)
    meanv = mean_v[...]
    c0 = plsc.load_gather(counts_v, [lane, zeros])
    c1 = plsc.load_gather(counts_v, [lane + 16, zeros])
    s0 = plsc.cumsum(c0)
    s1 = plsc.cumsum(c1)
    tot0 = jnp.max(s0)
    total = tot0 + jnp.max(s1)
    e0 = s0 - c0
    e1 = (s1 - c1) + tot0
    lsel = jnp.where(w < 16, w, w - 16)
    ew = jnp.where(w < 16, e0, e1)
    cwv = jnp.where(w < 16, c0, c1)
    p_w = jnp.sum(jnp.where(lane == lsel, ew, 0))
    c_w = jnp.sum(jnp.where(lane == lsel, cwv, 0))
    h = jnp.mod(8 - jnp.mod(p_w, 8), 8)
    a0 = p_w + h

    def outer(f, wp):
        pltpu.sync_copy(loss_hbm.at[pl.ds(base + f * S, S)], loss_v)
        pltpu.sync_copy(x_hbm.at[pl.ds(base + f * S, S)], x_v)
        pltpu.sync_copy(t_hbm.at[pl.ds(base + f * S, S)], t_v)

        def inner(p, wpv):
            sl = pl.ds(p * 16, 16)
            m = loss_v[sl] > meanv
            xv = x_v[sl]
            tv = t_v[sl]
            pos = plsc.cumsum(m.astype(jnp.int32))
            tgt = wpv + pos - 1 - h
            mk = jnp.logical_and(m, tgt >= 0)
            mh = jnp.logical_and(m, tgt < 0)
            plsc.store_scatter(xk_v, [tgt], xv, mask=mk)
            plsc.store_scatter(tk_v, [tgt], tv, mask=mk)
            plsc.store_scatter(headx, [tgt + h], xv, mask=mh)
            plsc.store_scatter(headt, [tgt + h], tv, mask=mh)
            return wpv + plsc.all_reduce_population_count(m)

        return lax.fori_loop(0, S // 16, inner, wp)

    lax.fori_loop(0, C // S, outer, jnp.zeros((16,), jnp.int32))

    nfull = jnp.maximum(c_w - h, 0) // B

    def chunk_body(m, _):
        dst = pl.multiple_of(a0 + m * B, 8)
        dx = pltpu.async_copy(xk_v.at[pl.ds(m * B, B)],
                              xo_hbm.at[pl.ds(dst, B)], sem)
        dt = pltpu.async_copy(tk_v.at[pl.ds(m * B, B)],
                              to_hbm.at[pl.ds(dst, B)], sem)
        dx.wait()
        dt.wait()
        return 0

    lax.fori_loop(0, nfull, chunk_body, 0)

    hm = lane < jnp.minimum(h, c_w)
    ridx[0, pl.ds(0, 16)] = jnp.where(hm, p_w + lane, TRASH + lane)
    rdx[0, pl.ds(0, 16)] = headx[...]
    rdt[0, pl.ds(0, 16)] = headt[...]
    for p in range(1, G // 16):
        ridx[0, pl.ds(p * 16, 16)] = TRASH + lane
    d1 = pltpu.async_copy(rdx.at[0], xo_hbm.at[ridx.at[0]], sem)
    d2 = pltpu.async_copy(rdt.at[0], to_hbm.at[ridx.at[0]], sem)
    d1.wait()
    d2.wait()

    tbase = nfull * B

    def trow_body(r, _):
        @pl.when(tbase + r * G < c_w - h)
        def _():
            for p in range(G // 16):
                slot = tbase + r * G + p * 16
                sv = slot + lane
                valid = sv < c_w - h
                ridx[1, pl.ds(p * 16, 16)] = jnp.where(
                    valid, a0 + sv, TRASH + lane)
                rdx[1, pl.ds(p * 16, 16)] = xk_v[pl.ds(slot, 16)]
                rdt[1, pl.ds(p * 16, 16)] = tk_v[pl.ds(slot, 16)]
            e1_ = pltpu.async_copy(rdx.at[1], xo_hbm.at[ridx.at[1]], sem)
            e2_ = pltpu.async_copy(rdt.at[1], to_hbm.at[ridx.at[1]], sem)
            e1_.wait()
            e2_.wait()
        return 0

    lax.fori_loop(0, NTR, trow_body, 0)

    def tail_body(q, _):
        bs = base + q * B

        @pl.when(bs >= total)
        def _pure():
            off = bs - total
            ph = jnp.mod(off, 8)
            woff = pl.multiple_of(off - ph, 8)
            gx = pltpu.async_copy(xn_hbm.at[pl.ds(woff, B + 16)],
                                  winx, sem)
            gt = pltpu.async_copy(tn_hbm.at[pl.ds(woff, B + 16)],
                                  wint, sem)
            gx.wait()
            gt.wait()
            for i in range(B // 16):
                gi = (ph + i * 16) + lane
                xob[pl.ds(i * 16, 16)] = plsc.load_gather(winx, [gi])
                tob[pl.ds(i * 16, 16)] = plsc.load_gather(wint, [gi])
            wx = pltpu.async_copy(xob, xo_hbm.at[pl.ds(bs, B)], sem)
            wt = pltpu.async_copy(tob, to_hbm.at[pl.ds(bs, B)], sem)
            wx.wait()
            wt.wait()

        @pl.when(jnp.logical_and(bs < total, bs + B > total))
        def _boundary():
            gx = pltpu.async_copy(xn_hbm.at[pl.ds(0, B + 16)], winx, sem)
            gt = pltpu.async_copy(tn_hbm.at[pl.ds(0, B + 16)], wint, sem)
            gx.wait()
            gt.wait()

            def brow_body(r, _):
                @pl.when(bs + r * G + G > total)
                def _():
                    for p in range(G // 16):
                        j = (bs + r * G + p * 16) + lane
                        tv = j >= total
                        gi = jnp.where(tv, j - total, 0)
                        ridx[2, pl.ds(p * 16, 16)] = jnp.where(
                            tv, j, TRASH + lane)
                        rdx[2, pl.ds(p * 16, 16)] = plsc.load_gather(
                            winx, [gi])
                        rdt[2, pl.ds(p * 16, 16)] = plsc.load_gather(
                            wint, [gi])
                    f1 = pltpu.async_copy(rdx.at[2],
                                          xo_hbm.at[ridx.at[2]], sem)
                    f2 = pltpu.async_copy(rdt.at[2],
                                          to_hbm.at[ridx.at[2]], sem)
                    f1.wait()
                    f2.wait()
                return 0

            lax.fori_loop(0, B // G, brow_body, 0)
        return 0

    lax.fori_loop(0, C // B, tail_body, 0)


def kernel(loss, x, t):
    mean = jnp.mean(loss)
    mean_arr = jnp.full((16,), mean, jnp.float32)
    lf = jnp.pad(loss.reshape(-1), (0, NPAD - N), constant_values=-1.0)
    xf = jnp.pad(x.reshape(-1), (0, NPAD - N))
    tf = jnp.pad(t.reshape(-1), (0, NPAD - N))
    kn = jax.random.split(jax.random.key(1), 2)
    xn = jax.random.uniform(kn[0], (N, 1), minval=X_LO, maxval=X_HI,
                            dtype=jnp.float32).reshape(-1)
    tn = jax.random.uniform(kn[1], (N, 1), minval=T_LO, maxval=T_HI,
                            dtype=jnp.float32).reshape(-1)
    xn = jnp.pad(xn, (0, NPAD + 32 - N))
    tn = jnp.pad(tn, (0, NPAD + 32 - N))
    counts = _k1_count(lf, mean_arr)
    xo, to = _k2_place(lf, xf, tf, xn, tn, mean_arr, counts)
    return (xo[:N, None], to[:N, None])
